# Initial kernel scaffold; baseline (speedup 1.0000x reference)
import functools

import jax
import jax.numpy as jnp
from jax.experimental import pallas as pl
from jax.experimental.pallas import tpu as pltpu

BM = 256
N_ITER = 18


def _fused_kernel(x_tile_ref, x_full_ref, w_ref, y_ref, stats_ref,
                  support_s, acc_s, *, K, NT, B):
    b = pl.program_id(0)
    i = pl.program_id(1)

    @pl.when(i == 0)
    def _():
        support_s[...] = jax.lax.dot_general(
            x_full_ref[0], w_ref[...], (((0,), (0,)), ((), ())),
            preferred_element_type=jnp.float32)

    @pl.when((b == 0) & (i == 0))
    def _():
        acc_s[...] = jnp.zeros_like(acc_s)

    S = jax.lax.dot_general(
        x_tile_ref[0], x_full_ref[0], (((0,), (0,)), ((), ())),
        preferred_element_type=jnp.float32)

    rowmax = jnp.max(S, axis=1, keepdims=True)
    rowmin = jnp.min(S, axis=1, keepdims=True)

    lo, hi = rowmin, rowmax
    for _ in range(N_ITER):
        mid = 0.5 * (lo + hi)
        cnt = jnp.sum((S >= mid).astype(jnp.float32), axis=1, keepdims=True)
        ge = cnt >= K
        lo = jnp.where(ge, mid, lo)
        hi = jnp.where(ge, hi, mid)
    thr = lo

    P = jnp.exp(S - rowmax)
    denom = jnp.sum(P, axis=1, keepdims=True)
    A = jnp.where(S >= thr, P, 0.0) / denom

    out = jax.lax.dot_general(
        A, support_s[...], (((1,), (0,)), ((), ())),
        preferred_element_type=jnp.float32)
    out = jnp.where(out >= 0, out, 0.01 * out)

    y_ref[0] = out
    acc_s[0:1, :] += jnp.sum(out, axis=0, keepdims=True)
    acc_s[1:2, :] += jnp.sum(out * out, axis=0, keepdims=True)

    @pl.when((b == B - 1) & (i == NT - 1))
    def _():
        stats_ref[...] = acc_s[...]


def _bn_kernel(y_ref, stats_ref, g_ref, bt_ref, o_ref, *, count):
    s = stats_ref[0:1, :]
    ss = stats_ref[1:2, :]
    mean = s / count
    var = ss / count - mean * mean
    inv = jax.lax.rsqrt(var + 1e-5)
    scale = g_ref[...] * inv
    shift = bt_ref[...] - mean * scale
    y = y_ref[0] * scale + shift
    o_ref[0] = y.T


def kernel(input, W, gamma, beta):
    B, C, N = input.shape
    O = W.shape[1]
    K = int(round(N * 0.9))
    NT = N // BM

    y, stats = pl.pallas_call(
        functools.partial(_fused_kernel, K=K, NT=NT, B=B),
        grid=(B, NT),
        in_specs=[
            pl.BlockSpec((1, C, BM), lambda b, i: (b, 0, i)),
            pl.BlockSpec((1, C, N), lambda b, i: (b, 0, 0)),
            pl.BlockSpec((C, O), lambda b, i: (0, 0)),
        ],
        out_specs=[
            pl.BlockSpec((1, BM, O), lambda b, i: (b, i, 0)),
            pl.BlockSpec((8, O), lambda b, i: (0, 0)),
        ],
        out_shape=[
            jax.ShapeDtypeStruct((B, N, O), jnp.float32),
            jax.ShapeDtypeStruct((8, O), jnp.float32),
        ],
        scratch_shapes=[
            pltpu.VMEM((N, O), jnp.float32),
            pltpu.VMEM((8, O), jnp.float32),
        ],
    )(input, input, W)

    BN2 = 2048
    out = pl.pallas_call(
        functools.partial(_bn_kernel, count=float(B * N)),
        grid=(B, N // BN2),
        in_specs=[
            pl.BlockSpec((1, BN2, O), lambda b, j: (b, j, 0)),
            pl.BlockSpec((8, O), lambda b, j: (0, 0)),
            pl.BlockSpec((1, O), lambda b, j: (0, 0)),
            pl.BlockSpec((1, O), lambda b, j: (0, 0)),
        ],
        out_specs=pl.BlockSpec((1, O, BN2), lambda b, j: (b, 0, j)),
        out_shape=jax.ShapeDtypeStruct((B, O, N), jnp.float32),
    )(y, stats, gamma.reshape(1, O), beta.reshape(1, O))
    return out


# speedup vs baseline: 626.3490x; 626.3490x over previous
import functools

import jax
import jax.numpy as jnp
from jax.experimental import pallas as pl
from jax.experimental.pallas import tpu as pltpu

BM = 512
N_ITER = 12


def _fused_kernel(x_tile_ref, x_full_ref, w_ref, y_ref, stats_ref,
                  support_s, acc_s, sbuf, *, K, NT, B):
    g = pl.program_id(0)
    G = B * NT + 1
    cur = jax.lax.rem(g, 2)

    @pl.when(g < G - 1)
    def _():
        sbuf[pl.ds(cur, 1)] = jax.lax.dot_general(
            x_tile_ref[0], x_full_ref[0], (((0,), (0,)), ((), ())),
            preferred_element_type=jnp.float32)[None]

    @pl.when((g >= 1) & (jax.lax.rem(g - 1, NT) == 0))
    def _():
        support_s[...] = jax.lax.dot_general(
            x_full_ref[0], w_ref[...], (((0,), (0,)), ((), ())),
            preferred_element_type=jnp.float32)

    @pl.when(g == 1)
    def _():
        acc_s[...] = jnp.zeros_like(acc_s)

    @pl.when(g >= 1)
    def _():
        S = sbuf[1 - cur]

        rowmax = jnp.max(S, axis=1, keepdims=True)
        rowmin = jnp.min(S, axis=1, keepdims=True)

        lo, hi = rowmin, rowmax
        for _ in range(N_ITER):
            mid = 0.5 * (lo + hi)
            cnt = jnp.sum((S >= mid).astype(jnp.float32), axis=1,
                          keepdims=True)
            ge = cnt >= K
            lo = jnp.where(ge, mid, lo)
            hi = jnp.where(ge, hi, mid)
        thr = lo

        P = jnp.exp(S - rowmax)
        denom = jnp.sum(P, axis=1, keepdims=True)
        A = jnp.where(S >= thr, P, 0.0)

        out = jax.lax.dot_general(
            A, support_s[...], (((1,), (0,)), ((), ())),
            preferred_element_type=jnp.float32)
        out = out / denom
        out = jnp.where(out >= 0, out, 0.01 * out)

        y_ref[0] = out
        acc_s[0:1, :] += jnp.sum(out, axis=0, keepdims=True)
        acc_s[1:2, :] += jnp.sum(out * out, axis=0, keepdims=True)

        @pl.when(g == G - 1)
        def _():
            stats_ref[...] = acc_s[...]


def _bn_kernel(y_ref, stats_ref, g_ref, bt_ref, o_ref, *, count):
    s = stats_ref[0:1, :]
    ss = stats_ref[1:2, :]
    mean = s / count
    var = ss / count - mean * mean
    inv = jax.lax.rsqrt(var + 1e-5)
    scale = g_ref[...] * inv
    shift = bt_ref[...] - mean * scale
    y = y_ref[0] * scale + shift
    o_ref[0] = y.T


def kernel(input, W, gamma, beta):
    B, C, N = input.shape
    O = W.shape[1]
    K = int(round(N * 0.9))
    NT = N // BM
    G = B * NT + 1
    last = B * NT - 1

    def comp_idx(g):
        f = jnp.minimum(g, last)
        return f // NT, f % NT

    def proc_idx(g):
        f = jnp.maximum(g - 1, 0)
        return f // NT, f % NT

    y, stats = pl.pallas_call(
        functools.partial(_fused_kernel, K=K, NT=NT, B=B),
        grid=(G,),
        in_specs=[
            pl.BlockSpec((1, C, BM),
                         lambda g: (comp_idx(g)[0], 0, comp_idx(g)[1])),
            pl.BlockSpec((1, C, N), lambda g: (comp_idx(g)[0], 0, 0)),
            pl.BlockSpec((C, O), lambda g: (0, 0)),
        ],
        out_specs=[
            pl.BlockSpec((1, BM, O),
                         lambda g: (proc_idx(g)[0], proc_idx(g)[1], 0)),
            pl.BlockSpec((8, O), lambda g: (0, 0)),
        ],
        out_shape=[
            jax.ShapeDtypeStruct((B, N, O), jnp.float32),
            jax.ShapeDtypeStruct((8, O), jnp.float32),
        ],
        scratch_shapes=[
            pltpu.VMEM((N, O), jnp.float32),
            pltpu.VMEM((8, O), jnp.float32),
            pltpu.VMEM((2, BM, N), jnp.float32),
        ],
    )(input, input, W)

    BN2 = 2048
    out = pl.pallas_call(
        functools.partial(_bn_kernel, count=float(B * N)),
        grid=(B, N // BN2),
        in_specs=[
            pl.BlockSpec((1, BN2, O), lambda b, j: (b, j, 0)),
            pl.BlockSpec((8, O), lambda b, j: (0, 0)),
            pl.BlockSpec((1, O), lambda b, j: (0, 0)),
            pl.BlockSpec((1, O), lambda b, j: (0, 0)),
        ],
        out_specs=pl.BlockSpec((1, O, BN2), lambda b, j: (b, 0, j)),
        out_shape=jax.ShapeDtypeStruct((B, O, N), jnp.float32),
    )(y, stats, gamma.reshape(1, O), beta.reshape(1, O))
    return out
